# baseline (speedup 1.0000x reference)
import functools

import jax
import jax.numpy as jnp
from jax import lax
from jax.experimental import pallas as pl
from jax.experimental.pallas import tpu as pltpu
from jax.experimental.pallas import tpu_sc as plsc

C = 100000
B = 1024
SMOOTH = 0.1
CONF = 1.0 - SMOOTH
SVAL = SMOOTH / (C - 1)

BR = 32
BV = 8192
KU = BV // 128
NR = B // BR
NV = (C + BV - 1) // BV
REM = C - (NV - 1) * BV
K_FULL_LAST = REM // 128
REM_LANES = REM - K_FULL_LAST * 128

SC_NC = 2
SC_NW = 32
BPW = B // SC_NW


def _sc_gather_body(x_hbm, t_hbm, out_hbm, t_v, idx_v, val_v, sem):
    wid = lax.axis_index("s") * SC_NC + lax.axis_index("c")
    base = wid * BPW
    pltpu.sync_copy(t_hbm.at[pl.ds(base, BPW)], t_v)
    for j in range(BPW // 16):
        t16 = t_v[pl.ds(j * 16, 16)]
        rows = base + j * 16 + lax.iota(jnp.int32, 16)
        idx_v[pl.ds(j * 16, 16)] = rows * C + t16
    pltpu.async_copy(x_hbm.at[idx_v], val_v, sem).wait()
    pltpu.sync_copy(val_v, out_hbm.at[pl.ds(base, BPW)])


@functools.partial(jax.jit, static_argnames=())
def _sc_gather(xflat, targets):
    k = functools.partial(
        pl.kernel,
        mesh=plsc.VectorSubcoreMesh(core_axis_name="c", subcore_axis_name="s"),
        out_type=jax.ShapeDtypeStruct((B,), jnp.float32),
        scratch_types=[
            pltpu.VMEM((BPW,), jnp.int32),
            pltpu.VMEM((BPW,), jnp.int32),
            pltpu.VMEM((BPW,), jnp.float32),
            pltpu.SemaphoreType.DMA,
        ],
    )(_sc_gather_body)
    return k(xflat, targets)


def _loss_body(g_ref, x_ref, o_ref, m_ref, s_ref, rs_ref):
    r = pl.program_id(0)
    v = pl.program_id(1)
    nv = pl.num_programs(1)

    @pl.when(v == 0)
    def _init():
        m_ref[...] = jnp.full((BR, 128), -jnp.inf, jnp.float32)
        s_ref[...] = jnp.zeros((BR, 128), jnp.float32)
        rs_ref[...] = jnp.zeros((BR, 128), jnp.float32)

    def update(nk, last_mask):
        m_old = m_ref[...]
        bmax = jnp.full((BR, 128), -jnp.inf, jnp.float32)
        rs = rs_ref[...]
        for k in range(nk):
            xk = x_ref[:, k * 128:(k + 1) * 128]
            if last_mask is not None and k == nk - 1:
                rs = rs + jnp.where(last_mask, xk, 0.0)
                xk = jnp.where(last_mask, xk, -jnp.inf)
            else:
                rs = rs + xk
            bmax = jnp.maximum(bmax, xk)
        rs_ref[...] = rs
        m_new = jnp.maximum(m_old, bmax)
        acc = s_ref[...] * jnp.exp(m_old - m_new)
        for k in range(nk):
            xk = x_ref[:, k * 128:(k + 1) * 128]
            if last_mask is not None and k == nk - 1:
                xk = jnp.where(last_mask, xk, -jnp.inf)
            acc = acc + jnp.exp(xk - m_new)
        s_ref[...] = acc
        m_ref[...] = m_new

    @pl.when(v < nv - 1)
    def _full():
        update(KU, None)

    @pl.when(v == nv - 1)
    def _last():
        if REM_LANES:
            lane = jax.lax.broadcasted_iota(jnp.int32, (BR, 128), 1)
            update(K_FULL_LAST + 1, lane < REM_LANES)
        else:
            update(K_FULL_LAST, None)
        m_acc = m_ref[...]
        m_row = jnp.max(m_acc, axis=1, keepdims=True)
        s_row = jnp.sum(s_ref[...] * jnp.exp(m_acc - m_row), axis=1,
                        keepdims=True)
        lse = m_row + jnp.log(s_row)
        rs_row = jnp.sum(rs_ref[...], axis=1, keepdims=True)
        g = g_ref[...]
        loss = -(SVAL * (rs_row - C * lse) + (CONF - SVAL) * (g - lse))
        part = jnp.reshape(jnp.sum(loss) / B, (1, 1))

        @pl.when(r == 0)
        def _():
            o_ref[...] = part

        @pl.when(r > 0)
        def _():
            o_ref[...] = o_ref[...] + part


def kernel(inputs, targets):
    g = _sc_gather(inputs.reshape(-1), targets)
    out = pl.pallas_call(
        _loss_body,
        grid=(NR, NV),
        in_specs=[
            pl.BlockSpec((BR, 1), lambda r, v: (r, 0)),
            pl.BlockSpec((BR, BV), lambda r, v: (r, v)),
        ],
        out_specs=pl.BlockSpec((1, 1), lambda r, v: (0, 0)),
        out_shape=jax.ShapeDtypeStruct((1, 1), jnp.float32),
        scratch_shapes=[pltpu.VMEM((BR, 128), jnp.float32) for _ in range(3)],
    )(g.reshape(B, 1), inputs)
    return out[0, 0]


# speedup vs baseline: 1.8144x; 1.8144x over previous
import functools

import jax
import jax.numpy as jnp
from jax import lax
from jax.experimental import pallas as pl
from jax.experimental.pallas import tpu as pltpu
from jax.experimental.pallas import tpu_sc as plsc

C = 100000
B = 1024
SMOOTH = 0.1
CONF = 1.0 - SMOOTH
SVAL = SMOOTH / (C - 1)

BR = 32
BV = 8192
KU = BV // 128
NR = B // BR
NV = (C + BV - 1) // BV
REM = C - (NV - 1) * BV
K_FULL_LAST = REM // 128
REM_LANES = REM - K_FULL_LAST * 128

SC_NC = 2
SC_NW = 32
BPW = B // SC_NW


def _sc_gather_body(x_hbm, t_hbm, out_hbm, t_v, idx_v, val_v, sem):
    wid = lax.axis_index("s") * SC_NC + lax.axis_index("c")
    base = wid * BPW
    pltpu.sync_copy(t_hbm.at[pl.ds(base, BPW)], t_v)
    for j in range(BPW // 16):
        t16 = t_v[pl.ds(j * 16, 16)]
        rows = base + j * 16 + lax.iota(jnp.int32, 16)
        idx_v[pl.ds(j * 16, 16)] = rows * C + t16
    pltpu.async_copy(x_hbm.at[idx_v], val_v, sem).wait()
    pltpu.sync_copy(val_v, out_hbm.at[pl.ds(base, BPW)])


@functools.partial(jax.jit, static_argnames=())
def _sc_gather(xflat, targets):
    k = functools.partial(
        pl.kernel,
        mesh=plsc.VectorSubcoreMesh(core_axis_name="c", subcore_axis_name="s"),
        out_type=jax.ShapeDtypeStruct((B,), jnp.float32),
        scratch_types=[
            pltpu.VMEM((BPW,), jnp.int32),
            pltpu.VMEM((BPW,), jnp.int32),
            pltpu.VMEM((BPW,), jnp.float32),
            pltpu.SemaphoreType.DMA,
        ],
    )(_sc_gather_body)
    return k(xflat, targets)


def _loss_body(t_ref, x_ref, o_ref, m_ref, s_ref, rs_ref, tg_ref):
    r = pl.program_id(0)
    v = pl.program_id(1)
    nv = pl.num_programs(1)

    @pl.when(v == 0)
    def _init():
        m_ref[...] = jnp.full((BR, 128), -jnp.inf, jnp.float32)
        s_ref[...] = jnp.zeros((BR, 128), jnp.float32)
        rs_ref[...] = jnp.zeros((BR, 128), jnp.float32)
        tg_ref[...] = jnp.zeros((BR, 128), jnp.float32)

    lane = jax.lax.broadcasted_iota(jnp.int32, (BR, 128), 1)

    def update(nk, last_mask):
        m_old = m_ref[...]
        bmax = jnp.full((BR, 128), -jnp.inf, jnp.float32)
        rs = rs_ref[...]
        for k in range(nk):
            xk = x_ref[:, k * 128:(k + 1) * 128]
            if last_mask is not None and k == nk - 1:
                rs = rs + jnp.where(last_mask, xk, 0.0)
                xk = jnp.where(last_mask, xk, -jnp.inf)
            else:
                rs = rs + xk
            bmax = jnp.maximum(bmax, xk)
        rs_ref[...] = rs
        m_new = jnp.maximum(m_old, bmax)
        rel = t_ref[...] - v * BV
        acc = s_ref[...] * jnp.exp(m_old - m_new)
        tg = tg_ref[...]
        for k in range(nk):
            xk = x_ref[:, k * 128:(k + 1) * 128]
            if last_mask is not None and k == nk - 1:
                xk = jnp.where(last_mask, xk, -jnp.inf)
            tg = tg + jnp.where(lane == rel - k * 128, xk, 0.0)
            acc = acc + jnp.exp(xk - m_new)
        s_ref[...] = acc
        tg_ref[...] = tg
        m_ref[...] = m_new

    @pl.when(v < nv - 1)
    def _full():
        update(KU, None)

    @pl.when(v == nv - 1)
    def _last():
        if REM_LANES:
            update(K_FULL_LAST + 1, lane < REM_LANES)
        else:
            update(K_FULL_LAST, None)
        m_acc = m_ref[...]
        m_row = jnp.max(m_acc, axis=1, keepdims=True)
        s_row = jnp.sum(s_ref[...] * jnp.exp(m_acc - m_row), axis=1,
                        keepdims=True)
        lse = m_row + jnp.log(s_row)
        rs_row = jnp.sum(rs_ref[...], axis=1, keepdims=True)
        g = jnp.sum(tg_ref[...], axis=1, keepdims=True)
        loss = -(SVAL * (rs_row - C * lse) + (CONF - SVAL) * (g - lse))
        part = jnp.reshape(jnp.sum(loss) / B, (1, 1))

        @pl.when(r == 0)
        def _():
            o_ref[...] = part

        @pl.when(r > 0)
        def _():
            o_ref[...] = o_ref[...] + part


def kernel(inputs, targets):
    t2 = targets.reshape(B, 1)
    out = pl.pallas_call(
        _loss_body,
        grid=(NR, NV),
        in_specs=[
            pl.BlockSpec((BR, 1), lambda r, v: (r, 0)),
            pl.BlockSpec((BR, BV), lambda r, v: (r, v)),
        ],
        out_specs=pl.BlockSpec((1, 1), lambda r, v: (0, 0)),
        out_shape=jax.ShapeDtypeStruct((1, 1), jnp.float32),
        scratch_shapes=[pltpu.VMEM((BR, 128), jnp.float32) for _ in range(4)],
    )(t2, inputs)
    return out[0, 0]


# speedup vs baseline: 4.2119x; 2.3213x over previous
import jax
import jax.numpy as jnp
from jax.experimental import pallas as pl
from jax.experimental.pallas import tpu as pltpu

C = 100000
B = 1024
SMOOTH = 0.1
CONF = 1.0 - SMOOTH
SVAL = SMOOTH / (C - 1)

BV = 1000
NV = C // BV
NG = BV // 8


def _loss_body(t_ref, x_ref, o_ref, m_ref, s_ref, rs_ref, tg_ref):
    v = pl.program_id(0)
    nv = pl.num_programs(0)

    @pl.when(v == 0)
    def _init():
        m_ref[...] = jnp.full((8, B), -jnp.inf, jnp.float32)
        s_ref[...] = jnp.zeros((8, B), jnp.float32)
        rs_ref[...] = jnp.zeros((8, B), jnp.float32)
        tg_ref[...] = jnp.zeros((8, B), jnp.float32)

    sub = jax.lax.broadcasted_iota(jnp.int32, (8, B), 0)
    trel = t_ref[...] - v * BV + jnp.zeros((8, B), jnp.int32)

    def p1(g, carry):
        bmax, rs = carry
        xg = x_ref[pl.ds(pl.multiple_of(g * 8, 8), 8), :]
        return jnp.maximum(bmax, xg), rs + xg

    bmax, rs = jax.lax.fori_loop(
        0, NG, p1,
        (jnp.full((8, B), -jnp.inf, jnp.float32), rs_ref[...]))
    rs_ref[...] = rs
    m_old = m_ref[...]
    m_new = jnp.maximum(m_old, bmax)
    m_ref[...] = m_new

    def p2(g, carry):
        acc, tg = carry
        xg = x_ref[pl.ds(pl.multiple_of(g * 8, 8), 8), :]
        hit = trel - g * 8 == sub
        return acc + jnp.exp(xg - m_new), tg + jnp.where(hit, xg, 0.0)

    acc, tg = jax.lax.fori_loop(
        0, NG, p2, (s_ref[...] * jnp.exp(m_old - m_new), tg_ref[...]))
    s_ref[...] = acc
    tg_ref[...] = tg

    @pl.when(v == nv - 1)
    def _fin():
        m_acc = m_ref[...]
        m_col = jnp.max(m_acc, axis=0, keepdims=True)
        s_col = jnp.sum(s_ref[...] * jnp.exp(m_acc - m_col), axis=0,
                        keepdims=True)
        lse = m_col + jnp.log(s_col)
        rs_col = jnp.sum(rs_ref[...], axis=0, keepdims=True)
        g_col = jnp.sum(tg_ref[...], axis=0, keepdims=True)
        loss = -(SVAL * (rs_col - C * lse) + (CONF - SVAL) * (g_col - lse))
        o_ref[...] = jnp.reshape(jnp.sum(loss) / B, (1, 1))


def kernel(inputs, targets):
    xt = inputs.T
    t2 = targets.reshape(1, B)
    out = pl.pallas_call(
        _loss_body,
        grid=(NV,),
        in_specs=[
            pl.BlockSpec((1, B), lambda v: (0, 0)),
            pl.BlockSpec((BV, B), lambda v: (v, 0)),
        ],
        out_specs=pl.BlockSpec((1, 1), lambda v: (0, 0)),
        out_shape=jax.ShapeDtypeStruct((1, 1), jnp.float32),
        scratch_shapes=[pltpu.VMEM((8, B), jnp.float32) for _ in range(4)],
    )(t2, xt)
    return out[0, 0]
